# Initial kernel scaffold; baseline (speedup 1.0000x reference)
import jax
import jax.numpy as jnp
from jax.experimental import pallas as pl


def kernel(x, conv_w, bn_gamma, bn_beta, bn_mean, bn_var):
    raise NotImplementedError("write your pallas kernel here")


# speedup vs baseline: 1.1158x; 1.1158x over previous
import jax
import jax.numpy as jnp
from jax.experimental import pallas as pl
from jax.experimental.pallas import tpu as pltpu


def _fused_kernel(x_ref, w_ref, shift_ref, o_ref):
    cin, hw = x_ref.shape[1], x_ref.shape[2]
    cout = o_ref.shape[1]
    n_chunks = hw // 128

    xb = x_ref[0]
    ps = xb[:, 0:128]
    for i in range(1, n_chunks):
        ps = ps + xb[:, i * 128:(i + 1) * 128]

    acc = jnp.dot(w_ref[...], ps,
                  preferred_element_type=jnp.float32)

    row = jnp.sum(acc, axis=1, keepdims=True)
    y = jnp.maximum(row + shift_ref[...], 0.0)

    tile = jnp.broadcast_to(y, (cout, 128))
    for i in range(n_chunks):
        o_ref[0, :, i * 128:(i + 1) * 128] = tile


def kernel(x, conv_w, bn_gamma, bn_beta, bn_mean, bn_var, eps=1e-5):
    n, cin, h, w = x.shape
    cout = conv_w.shape[0]
    hw = h * w

    scale = bn_gamma.astype(jnp.float32) / jnp.sqrt(
        bn_var.astype(jnp.float32) + eps)
    shift = (bn_beta.astype(jnp.float32)
             - bn_mean.astype(jnp.float32) * scale).reshape(cout, 1)
    w_folded = conv_w.astype(jnp.float32) * (scale[:, None] * (1.0 / hw))

    x_flat = x.reshape(n, cin, hw)

    out_flat = pl.pallas_call(
        _fused_kernel,
        out_shape=jax.ShapeDtypeStruct((n, cout, hw), x.dtype),
        grid=(n,),
        in_specs=[
            pl.BlockSpec((1, cin, hw), lambda b: (b, 0, 0)),
            pl.BlockSpec((cout, cin), lambda b: (0, 0)),
            pl.BlockSpec((cout, 1), lambda b: (0, 0)),
        ],
        out_specs=pl.BlockSpec((1, cout, hw), lambda b: (b, 0, 0)),
        compiler_params=pltpu.CompilerParams(
            dimension_semantics=("parallel",),
            vmem_limit_bytes=64 * 1024 * 1024,
        ),
    )(x_flat, w_folded, shift)

    return out_flat.reshape(n, cout, h, w)


# speedup vs baseline: 4.2219x; 3.7836x over previous
import jax
import jax.numpy as jnp
from jax.experimental import pallas as pl
from jax.experimental.pallas import tpu as pltpu


def _fused_kernel(x_ref, w_ref, shift_ref, o_ref):
    hw = x_ref.shape[1]
    cout = o_ref.shape[2]
    n_chunks = hw // 128

    xb = x_ref[0]
    ps = xb[0:128]
    for i in range(1, n_chunks):
        ps = ps + xb[i * 128:(i + 1) * 128]
    s = jnp.sum(ps, axis=0, keepdims=True)

    y = jnp.dot(s, w_ref[...],
                preferred_element_type=jnp.float32)
    y = jnp.maximum(y + shift_ref[...], 0.0)

    tile = jnp.broadcast_to(y, (128, cout))
    for i in range(n_chunks):
        o_ref[0, i * 128:(i + 1) * 128, :] = tile


def kernel(x, conv_w, bn_gamma, bn_beta, bn_mean, bn_var, eps=1e-5):
    n, cin, h, w = x.shape
    cout = conv_w.shape[0]
    hw = h * w

    scale = bn_gamma.astype(jnp.float32) / jnp.sqrt(
        bn_var.astype(jnp.float32) + eps)
    shift = (bn_beta.astype(jnp.float32)
             - bn_mean.astype(jnp.float32) * scale).reshape(1, cout)
    w_folded = (conv_w.astype(jnp.float32)
                * (scale[:, None] * (1.0 / hw))).T

    x_t = x.transpose(0, 2, 3, 1).reshape(n, hw, cin)

    out_t = pl.pallas_call(
        _fused_kernel,
        out_shape=jax.ShapeDtypeStruct((n, hw, cout), x.dtype),
        grid=(n,),
        in_specs=[
            pl.BlockSpec((1, hw, cin), lambda b: (b, 0, 0)),
            pl.BlockSpec((cin, cout), lambda b: (0, 0)),
            pl.BlockSpec((1, cout), lambda b: (0, 0)),
        ],
        out_specs=pl.BlockSpec((1, hw, cout), lambda b: (b, 0, 0)),
        compiler_params=pltpu.CompilerParams(
            dimension_semantics=("parallel",),
            vmem_limit_bytes=64 * 1024 * 1024,
        ),
    )(x_t, w_folded, shift)

    return out_t.reshape(n, h, w, cout).transpose(0, 3, 1, 2)
